# baseline (speedup 1.0000x reference)
import functools

import jax
import jax.numpy as jnp
from jax import lax
from jax.experimental import pallas as pl
from jax.experimental.pallas import tpu as pltpu
from jax.experimental.pallas import tpu_sc as plsc

_L = 16
_NS = 16
_ROWS = 2
_K = 3
_COLS = 1024
_W = _ROWS * _COLS // _NS
_NNZ = 4

_mesh = plsc.VectorSubcoreMesh(
    core_axis_name="c", subcore_axis_name="s", num_cores=1
)


@functools.partial(
    pl.kernel,
    out_type=jax.ShapeDtypeStruct((_ROWS * _COLS,), jnp.float32),
    mesh=_mesh,
    scratch_types=[
        pltpu.VMEM((_L,), jnp.int32),
        pltpu.VMEM((_L,), jnp.float32),
        pltpu.VMEM((_K, _W), jnp.float32),
        pltpu.VMEM((_W,), jnp.float32),
        pltpu.SemaphoreType.DMA,
    ],
)
def _coo_spmm(xind_hbm, xval_hbm, y_hbm, out_hbm, ind_v, val_v, y_v, o_v, sem):
    wid = lax.axis_index("s")
    base = wid * _W
    col = base % _COLS
    row = base // _COLS

    cps = [
        pltpu.async_copy(
            y_hbm.at[pl.ds(k * _COLS + col, _W)], y_v.at[k], sem
        )
        for k in range(_K)
    ]
    cps.append(pltpu.async_copy(xind_hbm, ind_v.at[pl.ds(0, 2 * _NNZ)], sem))
    cps.append(pltpu.async_copy(xval_hbm, val_v.at[pl.ds(0, _NNZ)], sem))
    for cp in cps:
        cp.wait()

    zero = jnp.float32(0.0)
    ind_vec = ind_v[...]
    vals_vec = val_v[...]
    coef = [[zero] * _K for _ in range(_ROWS)]
    for j in range(_NNZ):
        r = ind_vec[j]
        c = ind_vec[_NNZ + j]
        v = vals_vec[j]
        for i in range(_ROWS):
            for k in range(_K):
                coef[i][k] = coef[i][k] + jnp.where(
                    (r == i) & (c == k), v, zero
                )
    ck = [jnp.where(row == 0, coef[0][k], coef[1][k]) for k in range(_K)]

    for c0 in range(0, _W, _L):
        acc = ck[0] * y_v[0, pl.ds(c0, _L)]
        acc = acc + ck[1] * y_v[1, pl.ds(c0, _L)]
        acc = acc + ck[2] * y_v[2, pl.ds(c0, _L)]
        o_v[pl.ds(c0, _L)] = acc

    pltpu.async_copy(o_v, out_hbm.at[pl.ds(base, _W)], sem).wait()


def kernel(xind, xval, y):
    out_flat = _coo_spmm(xind.reshape(2 * _NNZ), xval, y.reshape(_K * _COLS))
    return out_flat.reshape(_ROWS, _COLS)
